# baseline (speedup 1.0000x reference)
import jax
import jax.numpy as jnp
from jax.experimental import pallas as pl
from jax.experimental.pallas import tpu as pltpu

B = 32
T = 4
H = 512
E = 8


def _gates(g):
    i = jax.nn.sigmoid(g[:, :H])
    f = jax.nn.sigmoid(g[:, H:2 * H])
    gg = jnp.tanh(g[:, 2 * H:3 * H])
    o = jax.nn.sigmoid(g[:, 3 * H:])
    return i, f, gg, o


def _lstm(gsum, b, c):
    i, f, gg, o = _gates(gsum + b)
    cn = f * c + i * gg
    hn = o * jnp.tanh(cn)
    return hn, cn


def _lstm0(gsum, b):
    i, _, gg, o = _gates(gsum + b)
    cn = i * gg
    hn = o * jnp.tanh(cn)
    return hn, cn


def _bf(a):
    return a.astype(jnp.bfloat16)


def _dot(a, b):
    return jax.lax.dot_general(_bf(a), b, (((1,), (0,)), ((), ())),
                               preferred_element_type=jnp.float32)


def _cat(a, b):
    return jnp.concatenate([a, b], axis=1)


def _moe_kernel(x0_ref, d_b1_ref, d_b2_ref, g_W_ref, g_b_ref,
                e_b1_ref, e_b2_ref, e_bo_ref,
                d_Wi1_h, d_Wh1_h, d_Wi2_h, d_Wh2_h,
                e_Wi1_h, e_Wh1_h, e_Wi2_h, e_Wh2_h, e_Wo_h,
                out_ref,
                st0, st1, st2, st3, st4, st5, st6, st7, st8,
                W1d, W2d, W2e, e_Wi1, e_Wh1, e_Wo,
                *sems):
    plan = (
        (d_Wi1_h, st0, W1d, 0),
        (d_Wi2_h, st1, W2d, 0),
        (e_Wi1_h, st2, e_Wi1, 0),
        (e_Wi2_h, st3, W2e, 0),
        (e_Wh1_h, st4, e_Wh1, 0),
        (e_Wh2_h, st5, W2e, H),
        (e_Wo_h, st6, e_Wo, 0),
        (d_Wh1_h, st7, W1d, H),
        (d_Wh2_h, st8, W2d, H),
    )
    copies = []
    for k, (src, st, bdst, off) in enumerate(plan):
        rows = src.shape[0]
        cols = src.shape[1]
        c = pltpu.make_async_copy(src, st.at[:rows, :cols], sems[k])
        c.start()
        copies.append((c, st, bdst, off, rows, cols))

    done = set()

    def rdy(bref, need_rows):
        for k, (c, st, bdst, off, rows, cols) in enumerate(copies):
            if bdst is bref and off < need_rows and k not in done:
                c.wait()
                bdst[off:off + rows, :cols] = _bf(st[:rows, :cols])
                done.add(k)
        return bref

    x0 = x0_ref[...]
    d_b1 = d_b1_ref[...]
    d_b2 = d_b2_ref[...]
    e_b1 = e_b1_ref[...]
    e_b2 = e_b2_ref[...]
    e_bo = e_bo_ref[...]
    g_b = g_b_ref[...]

    lane = jax.lax.broadcasted_iota(jnp.int32, (B, E), 1)

    def gate_weights(d_c2):
        logits = jax.lax.dot_general(
            d_c2, g_W_ref[...], (((1,), (0,)), ((), ())),
            preferred_element_type=jnp.float32) + g_b
        m = jnp.max(logits, axis=1, keepdims=True)
        ex = jnp.exp(logits - m)
        p = ex / jnp.sum(ex, axis=1, keepdims=True)
        m1 = jnp.max(p, axis=1, keepdims=True)
        i1 = jnp.min(jnp.where(p == m1, lane, E), axis=1, keepdims=True)
        p2 = jnp.where(lane == i1, -1.0, p)
        m2 = jnp.max(p2, axis=1, keepdims=True)
        i2 = jnp.min(jnp.where(p2 == m2, lane, E), axis=1, keepdims=True)
        w_lo = jnp.where(i1 < i2, m1, m2)
        w_hi = jnp.where(i1 < i2, m2, m1)
        return w_lo, w_hi

    d_h1, d_c1 = _lstm0(_dot(x0, rdy(W1d, H)[:H]), d_b1)
    d_h2, d_c2 = _lstm0(_dot(d_h1, rdy(W2d, H)[:H]), d_b2)
    w_lo, w_hi = gate_weights(d_c2)

    xw = _dot(x0, rdy(e_Wi1, H)[...])
    h1a, c1a = _lstm0(xw, e_b1)
    h2a, c2a = _lstm0(_dot(h1a, rdy(W2e, H)[:H]), e_b2)
    h1b, c1b = _lstm(xw + _dot(h1a, rdy(e_Wh1, H)[...]), e_b1, c1a)
    h2b, c2b = _lstm(_dot(_cat(h1b, h2a), rdy(W2e, 2 * H)[...]), e_b2, c2a)
    out_ab = _dot(jnp.concatenate([h2a, h2b], axis=0),
                  rdy(e_Wo, H)[...]) + e_bo
    e_h1, e_c1, e_h2, e_c2 = h1b, c1b, h2b, c2b

    o = w_lo * out_ab[:B] + w_hi * out_ab[B:]
    out_ref[:, 0:H] = o

    for t in range(1, T):
        xt = o
        xw = _dot(xt, e_Wi1[...])
        h1a, c1a = _lstm(xw + _dot(e_h1, e_Wh1[...]), e_b1, e_c1)
        h2a, c2a = _lstm(_dot(_cat(h1a, e_h2), W2e[...]), e_b2, e_c2)
        h1b, c1b = _lstm(xw + _dot(h1a, e_Wh1[...]), e_b1, c1a)
        h2b, c2b = _lstm(_dot(_cat(h1b, h2a), W2e[...]), e_b2, c2a)
        out_ab = _dot(jnp.concatenate([h2a, h2b], axis=0),
                      e_Wo[...]) + e_bo
        e_h1, e_c1, e_h2, e_c2 = h1b, c1b, h2b, c2b

        d_h1, d_c1 = _lstm(_dot(_cat(xt, d_h1), rdy(W1d, 2 * H)[...]),
                           d_b1, d_c1)
        d_h2, d_c2 = _lstm(_dot(_cat(d_h1, d_h2), rdy(W2d, 2 * H)[...]),
                           d_b2, d_c2)
        w_lo, w_hi = gate_weights(d_c2)

        o = w_lo * out_ab[:B] + w_hi * out_ab[B:]
        out_ref[:, t * H:(t + 1) * H] = o


def kernel(x, d_Wi1, d_Wh1, d_b1, d_Wi2, d_Wh2, d_b2, d_Wo, d_bo,
           g_W, g_b,
           e_Wi1, e_Wh1, e_b1, e_Wi2, e_Wh2, e_b2, e_Wo, e_bo):
    del d_Wo, d_bo
    x0 = x[:, 0, :]
    n_small = 8
    n_big = 9
    out = pl.pallas_call(
        _moe_kernel,
        out_shape=jax.ShapeDtypeStruct((B, T * H), jnp.float32),
        in_specs=(
            [pl.BlockSpec(memory_space=pltpu.MemorySpace.VMEM)] * n_small
            + [pl.BlockSpec(memory_space=pltpu.MemorySpace.HBM)] * n_big),
        out_specs=pl.BlockSpec(memory_space=pltpu.MemorySpace.VMEM),
        scratch_shapes=(
            [pltpu.VMEM((H, 4 * H), jnp.float32)] * 6
            + [pltpu.VMEM((H, H), jnp.float32)]
            + [pltpu.VMEM((H, 4 * H), jnp.float32)] * 2
            + [pltpu.VMEM((2 * H, 4 * H), jnp.bfloat16)] * 3
            + [pltpu.VMEM((H, 4 * H), jnp.bfloat16)] * 2
            + [pltpu.VMEM((H, H), jnp.bfloat16)]
            + [pltpu.SemaphoreType.DMA] * n_big),
    )(x0, d_b1.reshape(1, -1), d_b2.reshape(1, -1), g_W, g_b.reshape(1, -1),
      e_b1.reshape(1, -1), e_b2.reshape(1, -1), e_bo.reshape(1, -1),
      d_Wi1, d_Wh1, d_Wi2, d_Wh2, e_Wi1, e_Wh1, e_Wi2, e_Wh2, e_Wo)
    return out.reshape(B, T, H)
